# baseline (speedup 1.0000x reference)
import functools

import jax
import jax.numpy as jnp
from jax import lax
from jax.experimental import pallas as pl
from jax.experimental.pallas import tpu as pltpu
from jax.experimental.pallas import tpu_sc as plsc

B = 16384
C = 1000
A = 512
K = 10
NC = 2
NS = 16
NW = NC * NS
RB = B // NW
AB = A // NW
L = 16

TR = 512
TG = B // TR


def _sc_body(y_hbm, pos_hbm, neigh_hbm, w_out, pos_out,
             y_v, pos_v, neigh16_v, sp_v, wt_v):
    wid = lax.axis_index("s") * NC + lax.axis_index("c")
    base = wid * RB
    pltpu.sync_copy(y_hbm.at[pl.ds(base, RB)], y_v)
    pltpu.sync_copy(pos_hbm, pos_v.at[pl.ds(0, C)])
    pltpu.sync_copy(neigh_hbm.at[pl.ds(wid * AB * K, AB * K)],
                    neigh16_v.at[pl.ds(0, AB * K)])

    lane = lax.broadcasted_iota(jnp.int32, (L,), 0)

    def rows(j, _):
        off = j * L
        yv = y_v[pl.ds(off, L)]
        sp_v[pl.ds(off, L)] = plsc.load_gather(pos_v, [yv])
        return 0

    lax.fori_loop(0, RB // L, rows, 0)

    def zero(j, _):
        off = jnp.minimum(j * L, C - L)
        for a in range(AB):
            wt_v[a, pl.ds(off, L)] = jnp.zeros((L,), jnp.float32)
        return 0

    lax.fori_loop(0, (C + L - 1) // L, zero, 0)

    ones = jnp.ones((L,), jnp.float32)
    for k in range(K):
        nk = plsc.load_gather(neigh16_v, [lane * K + k])
        plsc.addupdate_scatter(wt_v, [lane, nk], ones)

    pltpu.sync_copy(wt_v, w_out.at[pl.ds(wid * AB, AB)])
    pltpu.sync_copy(sp_v, pos_out.at[wid])


def _sc_stage(y, pos, neigh):
    mesh = plsc.VectorSubcoreMesh(core_axis_name="c", subcore_axis_name="s",
                                  num_cores=NC, num_subcores=NS)
    f = functools.partial(
        pl.kernel, _sc_body, mesh=mesh,
        compiler_params=pltpu.CompilerParams(needs_layout_passes=False),
        out_type=[
            jax.ShapeDtypeStruct((A, C), jnp.float32),
            jax.ShapeDtypeStruct((NW, RB), jnp.int32),
        ],
        scratch_types=[
            pltpu.VMEM((RB,), jnp.int32),
            pltpu.VMEM((1024,), jnp.int32),
            pltpu.VMEM((256,), jnp.int32),
            pltpu.VMEM((RB,), jnp.int32),
            pltpu.VMEM((AB, C), jnp.float32),
        ],
    )()
    return f(y, pos, neigh)


def _tc_body(x_ref, y_ref, pos_ref, w_hbm, out_ref, w_vmem, wsem):
    pid = pl.program_id(0)

    @pl.when(pid == 0)
    def _():
        pltpu.make_async_copy(w_hbm, w_vmem, wsem).start()
        pltpu.make_async_copy(w_hbm, w_vmem, wsem).wait()

    xb = x_ref[...]
    yb = y_ref[0, 0, :]
    posb = pos_ref[0, 0, :]
    spb = jnp.maximum(posb, 0)
    maskb = jnp.where(posb >= 0, 1.0, 0.0)
    wbf = w_vmem[...].astype(jnp.bfloat16)

    m = jnp.max(xb, axis=1)
    e = jnp.exp(xb - m[:, None])

    acol = lax.broadcasted_iota(jnp.int32, (TR, A), 1)
    ohs = jnp.where(acol == spb[:, None], maskb[:, None], 0.0)
    v = lax.dot_general(ohs.astype(jnp.bfloat16), wbf,
                        (((1,), (0,)), ((), ())),
                        preferred_element_type=jnp.float32)
    col = lax.broadcasted_iota(jnp.int32, (TR, C), 1)
    u = v + jnp.where(col == yb[:, None], 1.0, 0.0)
    p = e * u

    ones2 = jnp.ones((C, 8), jnp.float32)
    s8 = lax.dot_general(e, ones2, (((1,), (0,)), ((), ())),
                         preferred_element_type=jnp.float32)
    n8 = lax.dot_general(p, ones2, (((1,), (0,)), ((), ())),
                         preferred_element_type=jnp.float32)
    loss = jnp.log(s8[:, 0]) - jnp.log(n8[:, 0])
    blk = jnp.sum(loss) * (1.0 / B)

    @pl.when(pid == 0)
    def _():
        out_ref[0, 0] = 0.0

    out_ref[0, 0] += blk


def kernel(x, y, ANs_position, ANs_neighbours):
    w, pg = _sc_stage(y, ANs_position, ANs_neighbours.reshape(A * K))
    y3 = y.reshape(TG, 1, TR)
    pg3 = pg.reshape(TG, 1, TR)
    out = pl.pallas_call(
        _tc_body,
        grid=(TG,),
        in_specs=[
            pl.BlockSpec((TR, C), lambda i: (i, 0)),
            pl.BlockSpec((1, 1, TR), lambda i: (i, 0, 0)),
            pl.BlockSpec((1, 1, TR), lambda i: (i, 0, 0)),
            pl.BlockSpec(memory_space=pltpu.MemorySpace.HBM),
        ],
        out_specs=pl.BlockSpec(memory_space=pltpu.MemorySpace.SMEM,
                               block_shape=(1, 1), index_map=lambda i: (0, 0)),
        out_shape=jax.ShapeDtypeStruct((1, 1), jnp.float32),
        compiler_params=pltpu.CompilerParams(
            dimension_semantics=("arbitrary",),
        ),
        scratch_shapes=[
            pltpu.VMEM((A, C), jnp.float32),
            pltpu.SemaphoreType.DMA,
        ],
    )(x, y3, pg3, w)
    return out[0, 0]


# speedup vs baseline: 1.0337x; 1.0337x over previous
import functools

import jax
import jax.numpy as jnp
from jax import lax
from jax.experimental import pallas as pl
from jax.experimental.pallas import tpu as pltpu
from jax.experimental.pallas import tpu_sc as plsc

B = 16384
C = 1000
A = 512
K = 10
NC = 2
NS = 16
NW = NC * NS
RB = B // NW
AB = A // NW
L = 16

TR = 2048
TG = B // TR


def _sc_body(y_hbm, pos_hbm, neigh_hbm, w_out, pos_out,
             y_v, pos_v, neigh16_v, sp_v, wt_v):
    wid = lax.axis_index("s") * NC + lax.axis_index("c")
    base = wid * RB
    pltpu.sync_copy(y_hbm.at[pl.ds(base, RB)], y_v)
    pltpu.sync_copy(pos_hbm, pos_v.at[pl.ds(0, C)])
    pltpu.sync_copy(neigh_hbm.at[pl.ds(wid * AB * K, AB * K)],
                    neigh16_v.at[pl.ds(0, AB * K)])

    lane = lax.broadcasted_iota(jnp.int32, (L,), 0)

    def rows(j, _):
        off = j * L
        yv = y_v[pl.ds(off, L)]
        sp_v[pl.ds(off, L)] = plsc.load_gather(pos_v, [yv])
        return 0

    lax.fori_loop(0, RB // L, rows, 0)

    def zero(j, _):
        off = jnp.minimum(j * L, C - L)
        for a in range(AB):
            wt_v[a, pl.ds(off, L)] = jnp.zeros((L,), jnp.float32)
        return 0

    lax.fori_loop(0, (C + L - 1) // L, zero, 0)

    ones = jnp.ones((L,), jnp.float32)
    for k in range(K):
        nk = plsc.load_gather(neigh16_v, [lane * K + k])
        plsc.addupdate_scatter(wt_v, [lane, nk], ones)

    pltpu.sync_copy(wt_v, w_out.at[pl.ds(wid * AB, AB)])
    pltpu.sync_copy(sp_v, pos_out.at[wid])


def _sc_stage(y, pos, neigh):
    mesh = plsc.VectorSubcoreMesh(core_axis_name="c", subcore_axis_name="s",
                                  num_cores=NC, num_subcores=NS)
    f = functools.partial(
        pl.kernel, _sc_body, mesh=mesh,
        compiler_params=pltpu.CompilerParams(needs_layout_passes=False),
        out_type=[
            jax.ShapeDtypeStruct((A, C), jnp.float32),
            jax.ShapeDtypeStruct((NW, RB), jnp.int32),
        ],
        scratch_types=[
            pltpu.VMEM((RB,), jnp.int32),
            pltpu.VMEM((1024,), jnp.int32),
            pltpu.VMEM((256,), jnp.int32),
            pltpu.VMEM((RB,), jnp.int32),
            pltpu.VMEM((AB, C), jnp.float32),
        ],
    )()
    return f(y, pos, neigh)


def _tc_body(x_ref, y_ref, pos_ref, w_hbm, out_ref, w_vmem, wsem):
    pid = pl.program_id(0)

    @pl.when(pid == 0)
    def _():
        pltpu.make_async_copy(w_hbm, w_vmem, wsem).start()
        pltpu.make_async_copy(w_hbm, w_vmem, wsem).wait()

    xb = x_ref[...]
    yb = y_ref[0, 0, :]
    posb = pos_ref[0, 0, :]
    spb = jnp.maximum(posb, 0)
    maskb = jnp.where(posb >= 0, 1.0, 0.0)
    wbf = w_vmem[...].astype(jnp.bfloat16)

    m = jnp.max(xb, axis=1)
    e = jnp.exp(xb - m[:, None])

    acol = lax.broadcasted_iota(jnp.int32, (TR, A), 1)
    ohs = jnp.where(acol == spb[:, None], maskb[:, None], 0.0)
    v = lax.dot_general(ohs.astype(jnp.bfloat16), wbf,
                        (((1,), (0,)), ((), ())),
                        preferred_element_type=jnp.float32)
    col = lax.broadcasted_iota(jnp.int32, (TR, C), 1)
    u = v + jnp.where(col == yb[:, None], 1.0, 0.0)
    p = e * u

    ones2 = jnp.ones((C, 8), jnp.float32)
    s8 = lax.dot_general(e, ones2, (((1,), (0,)), ((), ())),
                         preferred_element_type=jnp.float32)
    n8 = lax.dot_general(p, ones2, (((1,), (0,)), ((), ())),
                         preferred_element_type=jnp.float32)
    loss = jnp.log(s8[:, 0]) - jnp.log(n8[:, 0])
    blk = jnp.sum(loss) * (1.0 / B)

    @pl.when(pid == 0)
    def _():
        out_ref[0, 0] = 0.0

    out_ref[0, 0] += blk


def kernel(x, y, ANs_position, ANs_neighbours):
    w, pg = _sc_stage(y, ANs_position, ANs_neighbours.reshape(A * K))
    y3 = y.reshape(TG, 1, TR)
    pg3 = pg.reshape(TG, 1, TR)
    out = pl.pallas_call(
        _tc_body,
        grid=(TG,),
        in_specs=[
            pl.BlockSpec((TR, C), lambda i: (i, 0)),
            pl.BlockSpec((1, 1, TR), lambda i: (i, 0, 0)),
            pl.BlockSpec((1, 1, TR), lambda i: (i, 0, 0)),
            pl.BlockSpec(memory_space=pltpu.MemorySpace.HBM),
        ],
        out_specs=pl.BlockSpec(memory_space=pltpu.MemorySpace.SMEM,
                               block_shape=(1, 1), index_map=lambda i: (0, 0)),
        out_shape=jax.ShapeDtypeStruct((1, 1), jnp.float32),
        compiler_params=pltpu.CompilerParams(
            dimension_semantics=("arbitrary",),
        ),
        scratch_shapes=[
            pltpu.VMEM((A, C), jnp.float32),
            pltpu.SemaphoreType.DMA,
        ],
    )(x, y3, pg3, w)
    return out[0, 0]


# speedup vs baseline: 1.0463x; 1.0121x over previous
import functools

import jax
import jax.numpy as jnp
from jax import lax
from jax.experimental import pallas as pl
from jax.experimental.pallas import tpu as pltpu
from jax.experimental.pallas import tpu_sc as plsc

B = 16384
C = 1000
A = 512
K = 10
NC = 2
NS = 16
NW = NC * NS
RB = B // NW
AB = A // NW
L = 16

TR = 1024
TG = B // TR


def _sc_body(y_hbm, pos_hbm, neigh_hbm, w_out, pos_out,
             y_v, pos_v, neigh16_v, sp_v, wt_v):
    wid = lax.axis_index("s") * NC + lax.axis_index("c")
    base = wid * RB
    pltpu.sync_copy(y_hbm.at[pl.ds(base, RB)], y_v)
    pltpu.sync_copy(pos_hbm, pos_v.at[pl.ds(0, C)])
    pltpu.sync_copy(neigh_hbm.at[pl.ds(wid * AB * K, AB * K)],
                    neigh16_v.at[pl.ds(0, AB * K)])

    lane = lax.broadcasted_iota(jnp.int32, (L,), 0)

    def rows(j, _):
        off = j * L
        yv = y_v[pl.ds(off, L)]
        sp_v[pl.ds(off, L)] = plsc.load_gather(pos_v, [yv])
        return 0

    lax.fori_loop(0, RB // L, rows, 0)

    def zero(j, _):
        off = jnp.minimum(j * L, C - L)
        for a in range(AB):
            wt_v[a, pl.ds(off, L)] = jnp.zeros((L,), jnp.float32)
        return 0

    lax.fori_loop(0, (C + L - 1) // L, zero, 0)

    ones = jnp.ones((L,), jnp.float32)
    for k in range(K):
        nk = plsc.load_gather(neigh16_v, [lane * K + k])
        plsc.addupdate_scatter(wt_v, [lane, nk], ones)

    pltpu.sync_copy(wt_v, w_out.at[pl.ds(wid * AB, AB)])
    pltpu.sync_copy(sp_v, pos_out.at[wid])


def _sc_stage(y, pos, neigh):
    mesh = plsc.VectorSubcoreMesh(core_axis_name="c", subcore_axis_name="s",
                                  num_cores=NC, num_subcores=NS)
    f = functools.partial(
        pl.kernel, _sc_body, mesh=mesh,
        compiler_params=pltpu.CompilerParams(needs_layout_passes=False),
        out_type=[
            jax.ShapeDtypeStruct((A, C), jnp.float32),
            jax.ShapeDtypeStruct((NW, RB), jnp.int32),
        ],
        scratch_types=[
            pltpu.VMEM((RB,), jnp.int32),
            pltpu.VMEM((1024,), jnp.int32),
            pltpu.VMEM((256,), jnp.int32),
            pltpu.VMEM((RB,), jnp.int32),
            pltpu.VMEM((AB, C), jnp.float32),
        ],
    )()
    return f(y, pos, neigh)


def _tc_body(x_ref, y_ref, pos_ref, w_hbm, out_ref, w_vmem, wsem):
    pid = pl.program_id(0)

    @pl.when(pid == 0)
    def _():
        pltpu.make_async_copy(w_hbm, w_vmem, wsem).start()
        pltpu.make_async_copy(w_hbm, w_vmem, wsem).wait()

    xb = x_ref[...]
    yb = y_ref[0, 0, :]
    posb = pos_ref[0, 0, :]
    spb = jnp.maximum(posb, 0)
    maskb = jnp.where(posb >= 0, 1.0, 0.0)
    wbf = w_vmem[...].astype(jnp.bfloat16)

    m = jnp.max(xb, axis=1)
    e = jnp.exp(xb - m[:, None])

    acol = lax.broadcasted_iota(jnp.int32, (TR, A), 1)
    ohs = jnp.where(acol == spb[:, None], maskb[:, None], 0.0)
    v = lax.dot_general(ohs.astype(jnp.bfloat16), wbf,
                        (((1,), (0,)), ((), ())),
                        preferred_element_type=jnp.float32)
    col = lax.broadcasted_iota(jnp.int32, (TR, C), 1)
    u = v + jnp.where(col == yb[:, None], 1.0, 0.0)
    p = e * u

    ones2 = jnp.ones((C, 8), jnp.float32)
    s8 = lax.dot_general(e, ones2, (((1,), (0,)), ((), ())),
                         preferred_element_type=jnp.float32)
    n8 = lax.dot_general(p, ones2, (((1,), (0,)), ((), ())),
                         preferred_element_type=jnp.float32)
    loss = jnp.log(s8[:, 0]) - jnp.log(n8[:, 0])
    blk = jnp.sum(loss) * (1.0 / B)

    @pl.when(pid == 0)
    def _():
        out_ref[0, 0] = 0.0

    out_ref[0, 0] += blk


def kernel(x, y, ANs_position, ANs_neighbours):
    w, pg = _sc_stage(y, ANs_position, ANs_neighbours.reshape(A * K))
    y3 = y.reshape(TG, 1, TR)
    pg3 = pg.reshape(TG, 1, TR)
    out = pl.pallas_call(
        _tc_body,
        grid=(TG,),
        in_specs=[
            pl.BlockSpec((TR, C), lambda i: (i, 0)),
            pl.BlockSpec((1, 1, TR), lambda i: (i, 0, 0)),
            pl.BlockSpec((1, 1, TR), lambda i: (i, 0, 0)),
            pl.BlockSpec(memory_space=pltpu.MemorySpace.HBM),
        ],
        out_specs=pl.BlockSpec(memory_space=pltpu.MemorySpace.SMEM,
                               block_shape=(1, 1), index_map=lambda i: (0, 0)),
        out_shape=jax.ShapeDtypeStruct((1, 1), jnp.float32),
        compiler_params=pltpu.CompilerParams(
            dimension_semantics=("arbitrary",),
        ),
        scratch_shapes=[
            pltpu.VMEM((A, C), jnp.float32),
            pltpu.SemaphoreType.DMA,
        ],
    )(x, y3, pg3, w)
    return out[0, 0]
